# baseline (speedup 1.0000x reference)
import functools

import jax
import jax.numpy as jnp
from jax import lax
from jax.experimental import pallas as pl
from jax.experimental.pallas import tpu as pltpu
from jax.experimental.pallas import tpu_sc as plsc

N, H, W, C = 8, 224, 224, 96
TOTAL = N * H * W * C
NWORK = 32
PERW = TOTAL // NWORK
CH = 2048
NCH = PERW // CH
NPART = 4096
NBIN = 1 << 20
HSZ = NBIN + 32768
DSZ = TOTAL + NPART * 128 + CH
CSZ = TOTAL + 256
BINMASK = NBIN - 1

_mesh = plsc.VectorSubcoreMesh(core_axis_name="c", subcore_axis_name="s")
_sc_params = pltpu.CompilerParams(needs_layout_passes=False)


def _keys(x16):
    return plsc.bitcast(x16 + 0.0, jnp.int32)


@functools.partial(
    pl.kernel,
    out_type=jax.ShapeDtypeStruct((NWORK * NPART,), jnp.int32),
    mesh=_mesh,
    compiler_params=_sc_params,
    scratch_types=[
        pltpu.VMEM((NPART,), jnp.int32),
        pltpu.VMEM((CH,), jnp.float32),
    ],
)
def _s1(x_ref, g_ref, hist, xin):
    c = lax.axis_index("c")
    s = lax.axis_index("s")
    wid = c * 16 + s
    base = wid * PERW

    def zero(i, _):
        hist[pl.ds(i * 16, 16)] = jnp.zeros((16,), jnp.int32)
        return 0
    lax.fori_loop(0, NPART // 16, zero, 0)

    def chunk(j, _):
        pltpu.sync_copy(x_ref.at[pl.ds(base + j * CH, CH)], xin)

        def vec(i, _):
            k = _keys(xin[pl.ds(i * 16, 16)])
            p = lax.shift_right_logical(k, 20)
            r, last = plsc.scan_count(p)
            plsc.addupdate_scatter(hist, [p], r.astype(jnp.int32), mask=last)
            return 0
        lax.fori_loop(0, CH // 16, vec, 0)
        return 0
    lax.fori_loop(0, NCH, chunk, 0)
    pltpu.sync_copy(hist, g_ref.at[pl.ds(wid * NPART, NPART)])


def _cumsum_last(t, n):
    sh = 1
    while sh < n:
        z = jnp.zeros(t.shape[:-1] + (sh,), t.dtype)
        t = t + jnp.concatenate([z, t[..., :-sh]], axis=-1)
        sh *= 2
    return t


def _s2_body(g_ref, b_ref, o_ref, t_ref):
    g = g_ref[...]
    tot = jnp.sum(g, axis=0, keepdims=True)
    tpad = (tot + 127) & ~127
    o = _cumsum_last(tpad, NPART) - tpad
    sh = 1
    acc = g
    while sh < NWORK:
        z = jnp.zeros((sh, NPART), jnp.int32)
        acc = acc + jnp.concatenate([z, acc[:-sh]], axis=0)
        sh *= 2
    b_ref[...] = acc - g + o
    o_ref[...] = o
    t_ref[...] = tot


def _s2(g):
    return pl.pallas_call(
        _s2_body,
        out_shape=[
            jax.ShapeDtypeStruct((NWORK, NPART), jnp.int32),
            jax.ShapeDtypeStruct((1, NPART), jnp.int32),
            jax.ShapeDtypeStruct((1, NPART), jnp.int32),
        ],
    )(g)


@functools.partial(
    pl.kernel,
    out_type=[
        jax.ShapeDtypeStruct((DSZ,), jnp.int32),
        jax.ShapeDtypeStruct((DSZ,), jnp.int32),
    ],
    mesh=_mesh,
    compiler_params=_sc_params,
    scratch_types=[
        pltpu.VMEM((NPART,), jnp.int32),
        pltpu.VMEM((CH,), jnp.float32),
        pltpu.VMEM((CH,), jnp.int32),
        pltpu.VMEM((CH,), jnp.int32),
        pltpu.VMEM((16, 128), jnp.int32),
        pltpu.SemaphoreType.DMA,
    ],
)
def _s3(x_ref, bases_ref, dbin_ref, dpos_ref, cursor, xin, binb, posb, idx2d,
        sem):
    c = lax.axis_index("c")
    s = lax.axis_index("s")
    wid = c * 16 + s
    base = wid * PERW
    lanes = lax.iota(jnp.int32, 16)

    pltpu.sync_copy(bases_ref.at[pl.ds(wid * NPART, NPART)], cursor)

    def chunk(j, _):
        pltpu.sync_copy(x_ref.at[pl.ds(base + j * CH, CH)], xin)

        def vec(i, _):
            k = _keys(xin[pl.ds(i * 16, 16)])
            p = lax.shift_right_logical(k, 20)
            b = k & BINMASK
            r, last = plsc.scan_count(p)
            r = r.astype(jnp.int32)
            cur = plsc.load_gather(cursor, [p])
            dest = cur + r - 1
            plsc.addupdate_scatter(cursor, [p], r, mask=last)
            row = lax.shift_right_logical(i, 3)
            col = (i & 7) * 16
            idx2d[row, pl.ds(col, 16)] = dest
            binb[pl.ds(i * 16, 16)] = b
            posb[pl.ds(i * 16, 16)] = base + j * CH + i * 16 + lanes
            return 0
        lax.fori_loop(0, CH // 16, vec, 0)

        copies = []
        for row in range(16):
            copies.append(pltpu.async_copy(
                binb.at[pl.ds(row * 128, 128)],
                dbin_ref.at[idx2d.at[row]], sem))
            copies.append(pltpu.async_copy(
                posb.at[pl.ds(row * 128, 128)],
                dpos_ref.at[idx2d.at[row]], sem))
        for cp in copies:
            cp.wait()
        return 0
    lax.fori_loop(0, NCH, chunk, 0)


@functools.partial(
    pl.kernel,
    out_type=jax.ShapeDtypeStruct((CSZ,), jnp.int32),
    mesh=_mesh,
    compiler_params=_sc_params,
    scratch_types=[
        pltpu.VMEM_SHARED((HSZ,), jnp.int32),
        pltpu.VMEM((NPART,), jnp.int32),
        pltpu.VMEM((NPART,), jnp.int32),
        pltpu.VMEM((NPART // 2,), jnp.int32),
        pltpu.VMEM((16,), jnp.int32),
        pltpu.VMEM((CH,), jnp.int32),
        pltpu.VMEM((CH,), jnp.int32),
        pltpu.VMEM((CH,), jnp.int32),
        pltpu.VMEM((CH,), jnp.int32),
        pltpu.VMEM((CH,), jnp.int32),
        pltpu.VMEM((16, 128), jnp.int32),
        pltpu.VMEM((16, 128), jnp.int32),
        pltpu.SemaphoreType.DMA,
    ],
)
def _s4(dbin_ref, dpos_ref, o_ref, t_ref, ev_ref, od_ref, meta_ref,
        c_out, hist, ovm, tvm, occvm, stepsvm, binin, posin, cnt, ones,
        zeros, bidx, pidx, sem):
    c = lax.axis_index("c")
    s = lax.axis_index("s")
    lanes = lax.iota(jnp.int32, 16)
    dump = NBIN + s * 16 + lanes
    sink = TOTAL + s * 16 + lanes

    pltpu.sync_copy(o_ref, ovm)
    pltpu.sync_copy(t_ref, tvm)

    @pl.when(c == 0)
    def _():
        pltpu.sync_copy(ev_ref, occvm)

    @pl.when(c != 0)
    def _():
        pltpu.sync_copy(od_ref, occvm)

    pltpu.sync_copy(meta_ref, stepsvm)
    steps = stepsvm[pl.ds(0, 16)][0]

    def fill(i, _):
        ones[pl.ds(i * 16, 16)] = jnp.full((16,), 1, jnp.int32)
        zeros[pl.ds(i * 16, 16)] = jnp.zeros((16,), jnp.int32)
        return 0
    lax.fori_loop(0, CH // 16, fill, 0)

    nstripe = HSZ // 16 // CH

    def clear(j, _):
        pltpu.sync_copy(zeros, hist.at[pl.ds((s * nstripe + j) * CH, CH)])
        return 0
    lax.fori_loop(0, nstripe, clear, 0)
    plsc.subcore_barrier()

    def fix_chunk(cbase, en):
        cbase = pl.multiple_of(cbase, 8)
        pltpu.sync_copy(dbin_ref.at[pl.ds(cbase, CH)], binin)
        pltpu.sync_copy(dpos_ref.at[pl.ds(cbase, CH)], posin)

        def vec(i, _):
            valid = (cbase + i * 16 + lanes) < en
            b = jnp.where(valid, binin[pl.ds(i * 16, 16)], dump)
            q = jnp.where(valid, posin[pl.ds(i * 16, 16)], sink)
            row = lax.shift_right_logical(i, 3)
            col = (i & 7) * 16
            bidx[row, pl.ds(col, 16)] = b
            pidx[row, pl.ds(col, 16)] = q
            return 0
        lax.fori_loop(0, CH // 16, vec, 0)

    def step_fn(t, _):
        tv = jnp.full((16,), t, jnp.int32)
        p = plsc.load_gather(occvm, [tv])[0]
        pv = jnp.full((16,), p, jnp.int32)
        o = plsc.load_gather(ovm, [pv])[0]
        n = plsc.load_gather(tvm, [pv])[0]
        slice_sz = ((lax.shift_right_logical(n + 15, 4) + 7) // 8) * 8
        st = o + s * slice_sz
        en = jnp.minimum(st + slice_sz, o + n)
        nch = lax.shift_right_logical(jnp.maximum(en - st, 0) + CH - 1, 11)

        def add_chunk(cc, _):
            fix_chunk(st + cc * CH, en)
            copies = [pltpu.async_copy(ones.at[pl.ds(row * 128, 128)],
                                       hist.at[bidx.at[row]], sem, add=True)
                      for row in range(16)]
            for cp in copies:
                cp.wait()
            return 0
        lax.fori_loop(0, nch, add_chunk, 0)
        plsc.subcore_barrier()

        def read_chunk(cc, _):
            fix_chunk(st + cc * CH, en)
            copies = [pltpu.async_copy(hist.at[bidx.at[row]],
                                       cnt.at[pl.ds(row * 128, 128)], sem)
                      for row in range(16)]
            for cp in copies:
                cp.wait()
            copies = [pltpu.async_copy(cnt.at[pl.ds(row * 128, 128)],
                                       c_out.at[pidx.at[row]], sem)
                      for row in range(16)]
            for cp in copies:
                cp.wait()
            return 0
        lax.fori_loop(0, nch, read_chunk, 0)
        plsc.subcore_barrier()

        def zero_chunk(cc, _):
            fix_chunk(st + cc * CH, en)
            copies = [pltpu.async_copy(zeros.at[pl.ds(row * 128, 128)],
                                       hist.at[bidx.at[row]], sem)
                      for row in range(16)]
            for cp in copies:
                cp.wait()
            return 0
        lax.fori_loop(0, nch, zero_chunk, 0)
        plsc.subcore_barrier()
        return 0
    lax.fori_loop(0, steps, step_fn, 0)


def _s5_body(x_ref, c_ref, o_ref):
    xa = x_ref[0, 0, :, 0:96]
    xb = x_ref[0, 0, :, 96:192]
    xc = x_ref[0, 1, :, 0:96]
    xd = x_ref[0, 1, :, 96:192]
    ca = c_ref[0, 0, :, 0:96]
    cb = c_ref[0, 0, :, 96:192]
    cc = c_ref[0, 1, :, 0:96]
    cd = c_ref[0, 1, :, 96:192]
    v, m = xa, ca
    for cn, xn in ((cb, xb), (cc, xc), (cd, xd)):
        t = cn < m
        v = jnp.where(t, xn, v)
        m = jnp.where(t, cn, m)
    o_ref[0, 0] = v


def _s5(xv, cv):
    return pl.pallas_call(
        _s5_body,
        grid=(N, H // 2),
        in_specs=[
            pl.BlockSpec((1, 2, W // 2, 2 * C), lambda n, i: (n, i, 0, 0)),
            pl.BlockSpec((1, 2, W // 2, 2 * C), lambda n, i: (n, i, 0, 0)),
        ],
        out_specs=pl.BlockSpec((1, 1, W // 2, C), lambda n, i: (n, i, 0, 0)),
        out_shape=jax.ShapeDtypeStruct((N, H // 2, W // 2, C), jnp.float32),
    )(xv, cv)


def kernel(inputs):
    xf = inputs.reshape(-1)
    g = _s1(xf)
    bases, o, t = _s2(g.reshape(NWORK, NPART))
    t1 = t.reshape(-1)
    occ = t1 > 0
    order = jnp.argsort(jnp.where(occ, 0, 1), stable=True).astype(jnp.int32)
    n_occ = jnp.sum(occ.astype(jnp.int32))
    steps = (n_occ + 1) // 2
    ev = order[0::2]
    od = order[1::2]
    meta = jnp.full((16,), steps, jnp.int32)
    dbin, dpos = _s3(xf, bases.reshape(-1))
    cnt = _s4(dbin, dpos, o.reshape(-1), t1, ev, od, meta)
    xv = inputs.reshape(N, H, W // 2, 2 * C)
    cv = cnt[:TOTAL].reshape(N, H, W // 2, 2 * C)
    return _s5(xv, cv)


# speedup vs baseline: 8.4253x; 8.4253x over previous
import functools

import jax
import jax.numpy as jnp
from jax import lax
from jax.experimental import pallas as pl
from jax.experimental.pallas import tpu as pltpu
from jax.experimental.pallas import tpu_sc as plsc

N, H, W, C = 8, 224, 224, 96
TOTAL = N * H * W * C
NWORK = 32
PERW = TOTAL // NWORK
CH = 2048
NCH = PERW // CH
NPART = 8192
NBIN = 1 << 19
HROWS = NBIN // 16
DSZ = TOTAL + NPART * 128 + 8192
CH4 = 8192
WAYS = 16
HV = 1 << 16
BINMASK = NBIN - 1

_mesh = plsc.VectorSubcoreMesh(core_axis_name="c", subcore_axis_name="s")
_sc_params = pltpu.CompilerParams(needs_layout_passes=False)


def _keys(x16):
    return plsc.bitcast(x16 + 0.0, jnp.int32)


@functools.partial(
    pl.kernel,
    out_type=jax.ShapeDtypeStruct((NWORK * NPART,), jnp.int32),
    mesh=_mesh,
    compiler_params=_sc_params,
    scratch_types=[
        pltpu.VMEM((NPART,), jnp.int32),
        pltpu.VMEM((CH,), jnp.float32),
    ],
)
def _s1(x_ref, g_ref, hist, xin):
    c = lax.axis_index("c")
    s = lax.axis_index("s")
    wid = c * 16 + s
    base = wid * PERW

    def zero(i, _):
        hist[pl.ds(i * 16, 16)] = jnp.zeros((16,), jnp.int32)
        return 0
    lax.fori_loop(0, NPART // 16, zero, 0)

    def chunk(j, _):
        pltpu.sync_copy(x_ref.at[pl.ds(base + j * CH, CH)], xin)

        def vec(i, _):
            k = _keys(xin[pl.ds(i * 16, 16)])
            p = lax.shift_right_logical(k, 19)
            r, last = plsc.scan_count(p)
            plsc.addupdate_scatter(hist, [p], r.astype(jnp.int32), mask=last)
            return 0
        lax.fori_loop(0, CH // 16, vec, 0)
        return 0
    lax.fori_loop(0, NCH, chunk, 0)
    pltpu.sync_copy(hist, g_ref.at[pl.ds(wid * NPART, NPART)])


def _cumsum_last(t, n):
    sh = 1
    while sh < n:
        z = jnp.zeros(t.shape[:-1] + (sh,), t.dtype)
        t = t + jnp.concatenate([z, t[..., :-sh]], axis=-1)
        sh *= 2
    return t


def _s2_body(g_ref, b_ref, o_ref, t_ref):
    g = g_ref[...]
    tot = jnp.sum(g, axis=0, keepdims=True)
    tpad = (tot + 127) & ~127
    o = _cumsum_last(tpad, NPART) - tpad
    sh = 1
    acc = g
    while sh < NWORK:
        z = jnp.zeros((sh, NPART), jnp.int32)
        acc = acc + jnp.concatenate([z, acc[:-sh]], axis=0)
        sh *= 2
    b_ref[...] = acc - g + o
    o_ref[...] = o
    t_ref[...] = tot


def _s2(g):
    return pl.pallas_call(
        _s2_body,
        out_shape=[
            jax.ShapeDtypeStruct((NWORK, NPART), jnp.int32),
            jax.ShapeDtypeStruct((1, NPART), jnp.int32),
            jax.ShapeDtypeStruct((1, NPART), jnp.int32),
        ],
    )(g)


@functools.partial(
    pl.kernel,
    out_type=[
        jax.ShapeDtypeStruct((DSZ,), jnp.int32),
        jax.ShapeDtypeStruct((TOTAL,), jnp.int32),
    ],
    mesh=_mesh,
    compiler_params=_sc_params,
    scratch_types=[
        pltpu.VMEM((NPART,), jnp.int32),
        pltpu.VMEM((CH,), jnp.float32),
        pltpu.VMEM((CH,), jnp.int32),
        pltpu.VMEM((CH,), jnp.int32),
        pltpu.SemaphoreType.DMA,
    ],
)
def _s3(x_ref, bases_ref, dbin_ref, ddest_ref, cursor, xin, binb, destb, sem):
    c = lax.axis_index("c")
    s = lax.axis_index("s")
    wid = c * 16 + s
    base = wid * PERW

    pltpu.sync_copy(bases_ref.at[pl.ds(wid * NPART, NPART)], cursor)

    def chunk(j, _):
        pltpu.sync_copy(x_ref.at[pl.ds(base + j * CH, CH)], xin)

        def vec(i, _):
            k = _keys(xin[pl.ds(i * 16, 16)])
            p = lax.shift_right_logical(k, 19)
            b = k & BINMASK
            r, last = plsc.scan_count(p)
            r = r.astype(jnp.int32)
            cur = plsc.load_gather(cursor, [p])
            dest = cur + r - 1
            plsc.addupdate_scatter(cursor, [p], r, mask=last)
            destb[pl.ds(i * 16, 16)] = dest
            binb[pl.ds(i * 16, 16)] = b
            return 0
        lax.fori_loop(0, CH // 16, vec, 0)

        c1 = pltpu.async_copy(binb, dbin_ref.at[destb], sem)
        c2 = pltpu.async_copy(destb, ddest_ref.at[pl.ds(base + j * CH, CH)],
                              sem)
        c1.wait()
        c2.wait()
        return 0
    lax.fori_loop(0, NCH, chunk, 0)


@functools.partial(
    pl.kernel,
    out_type=jax.ShapeDtypeStruct((NWORK * DSZ,), jnp.int32),
    mesh=_mesh,
    compiler_params=_sc_params,
    scratch_types=[
        pltpu.VMEM((HV,), jnp.int32),
        pltpu.VMEM((NPART,), jnp.int32),
        pltpu.VMEM((NPART,), jnp.int32),
        pltpu.VMEM((NPART,), jnp.int32),
        pltpu.VMEM((16,), jnp.int32),
        pltpu.VMEM((CH4,), jnp.int32),
        pltpu.VMEM((CH4,), jnp.int32),
        pltpu.SemaphoreType.DMA,
    ],
)
def _s4(dbin_ref, o_ref, t_ref, ev_ref, od_ref, meta_ref,
        cgw_ref, histv, ovm, tvm, occvm, stepsvm, binin, cnt, sem):
    c = lax.axis_index("c")
    s = lax.axis_index("s")
    lanes = lax.iota(jnp.int32, 16)
    wbase = (c * 16 + s) * DSZ

    pltpu.sync_copy(o_ref, ovm)
    pltpu.sync_copy(t_ref, tvm)

    @pl.when(c == 0)
    def _():
        pltpu.sync_copy(ev_ref, occvm)

    @pl.when(c != 0)
    def _():
        pltpu.sync_copy(od_ref, occvm)

    pltpu.sync_copy(meta_ref, stepsvm)
    sv = stepsvm[pl.ds(0, 16)]
    steps = jnp.where(c == 0, sv[0], sv[1])

    def zero_hist():
        def z(i, _):
            histv[pl.ds(i * 16, 16)] = jnp.zeros((16,), jnp.int32)
            return 0
        lax.fori_loop(0, HV // 16, z, 0)
    zero_hist()

    def step_fn(t, _):
        tv = jnp.full((16,), t, jnp.int32)
        p = plsc.load_gather(occvm, [tv])[0]
        pv = jnp.full((16,), p, jnp.int32)
        o = plsc.load_gather(ovm, [pv])[0]
        n = plsc.load_gather(tvm, [pv])[0]
        nch = lax.shift_right_logical(n + CH4 - 1, 13)
        nch = jnp.where(lax.shift_right_logical(p, 1) % 2 ==
                        lax.shift_right_logical(s, 3), nch, 0)
        en = o + n

        def add_chunk(cc, _):
            cbase = pl.multiple_of(o + cc * CH4, 8)
            pltpu.sync_copy(dbin_ref.at[pl.ds(cbase, CH4)], binin)

            def vec(i, _):
                b = binin[pl.ds(i * 16, 16)] & BINMASK
                valid = (cbase + i * 16 + lanes) < en
                mine = ((b & 7) == (s & 7)) & valid
                bk = jnp.where(mine, b, NBIN + lanes)
                r, last = plsc.scan_count(bk)
                hi = lax.shift_right_logical(b, 3)
                plsc.addupdate_scatter(histv, [hi], r.astype(jnp.int32),
                                       mask=last & mine)
                return 0
            lax.fori_loop(0, CH4 // 16, vec, 0)
            return 0
        lax.fori_loop(0, nch, add_chunk, 0)

        def read_chunk(cc, _):
            cbase = pl.multiple_of(o + cc * CH4, 8)
            pltpu.sync_copy(dbin_ref.at[pl.ds(cbase, CH4)], binin)

            def vec(i, _):
                b = binin[pl.ds(i * 16, 16)] & BINMASK
                valid = (cbase + i * 16 + lanes) < en
                mine = ((b & 7) == (s & 7)) & valid
                hi = lax.shift_right_logical(b, 3)
                cv = plsc.load_gather(histv, [hi])
                cnt[pl.ds(i * 16, 16)] = jnp.where(mine, cv, 0)
                return 0
            lax.fori_loop(0, CH4 // 16, vec, 0)
            pltpu.sync_copy(cnt, cgw_ref.at[pl.ds(wbase + cbase, CH4)])
            return 0
        lax.fori_loop(0, nch, read_chunk, 0)
        zero_hist()
        return 0
    lax.fori_loop(0, steps, step_fn, 0)


@functools.partial(
    pl.kernel,
    out_type=jax.ShapeDtypeStruct((TOTAL,), jnp.int32),
    mesh=_mesh,
    compiler_params=_sc_params,
    scratch_types=[
        pltpu.VMEM((CH,), jnp.float32),
        pltpu.VMEM((CH,), jnp.int32),
        pltpu.VMEM((CH,), jnp.int32),
        pltpu.VMEM((CH,), jnp.int32),
        pltpu.SemaphoreType.DMA,
    ],
)
def _f(x_ref, ddest_ref, cgw_ref, c_ref, xin, destin, gidx, cnt, sem):
    c = lax.axis_index("c")
    s = lax.axis_index("s")
    wid = c * 16 + s
    base = wid * PERW
    lanes = lax.iota(jnp.int32, 16)

    def chunk(j, _):
        pltpu.sync_copy(x_ref.at[pl.ds(base + j * CH, CH)], xin)
        pltpu.sync_copy(ddest_ref.at[pl.ds(base + j * CH, CH)], destin)

        def vec(i, _):
            k = _keys(xin[pl.ds(i * 16, 16)])
            d = destin[pl.ds(i * 16, 16)]
            way = k & 7
            core = lax.shift_right_logical(k, 19) & 1
            group = lax.shift_right_logical(k, 20) & 1
            owner = core * 16 + group * 8 + way
            gidx[pl.ds(i * 16, 16)] = owner * DSZ + d
            return 0
        lax.fori_loop(0, CH // 16, vec, 0)
        pltpu.async_copy(cgw_ref.at[gidx], cnt, sem).wait()
        pltpu.sync_copy(cnt, c_ref.at[pl.ds(base + j * CH, CH)])
        return 0
    lax.fori_loop(0, NCH, chunk, 0)


def _s5_body(x_ref, c_ref, o_ref):
    xa = x_ref[0, 0, :, 0:96]
    xb = x_ref[0, 0, :, 96:192]
    xc = x_ref[0, 1, :, 0:96]
    xd = x_ref[0, 1, :, 96:192]
    ca = c_ref[0, 0, :, 0:96]
    cb = c_ref[0, 0, :, 96:192]
    cc = c_ref[0, 1, :, 0:96]
    cd = c_ref[0, 1, :, 96:192]
    v, m = xa, ca
    for cn, xn in ((cb, xb), (cc, xc), (cd, xd)):
        t = cn < m
        v = jnp.where(t, xn, v)
        m = jnp.where(t, cn, m)
    o_ref[0, 0] = v


def _s5(xv, cv):
    return pl.pallas_call(
        _s5_body,
        grid=(N, H // 2),
        in_specs=[
            pl.BlockSpec((1, 2, W // 2, 2 * C), lambda n, i: (n, i, 0, 0)),
            pl.BlockSpec((1, 2, W // 2, 2 * C), lambda n, i: (n, i, 0, 0)),
        ],
        out_specs=pl.BlockSpec((1, 1, W // 2, C), lambda n, i: (n, i, 0, 0)),
        out_shape=jax.ShapeDtypeStruct((N, H // 2, W // 2, C), jnp.float32),
    )(xv, cv)


def kernel(inputs):
    xf = inputs.reshape(-1)
    g = _s1(xf)
    bases, o, t = _s2(g.reshape(NWORK, NPART))
    t1 = t.reshape(-1)
    ids = jnp.arange(NPART, dtype=jnp.int32)
    occ_ev = (t1 > 0) & (ids % 2 == 0)
    occ_od = (t1 > 0) & (ids % 2 == 1)
    ev = jnp.argsort(jnp.where(occ_ev, 0, 1), stable=True).astype(jnp.int32)
    od = jnp.argsort(jnp.where(occ_od, 0, 1), stable=True).astype(jnp.int32)
    steps_ev = jnp.sum(occ_ev.astype(jnp.int32))
    steps_od = jnp.sum(occ_od.astype(jnp.int32))
    meta = jnp.stack([steps_ev, steps_od]).astype(jnp.int32)
    meta = jnp.concatenate([meta, jnp.zeros((14,), jnp.int32)])
    dbin, ddest = _s3(xf, bases.reshape(-1))
    cgw = _s4(dbin, o.reshape(-1), t1, ev, od, meta)
    cnt = _f(xf, ddest, cgw)
    xv = inputs.reshape(N, H, W // 2, 2 * C)
    cv = cnt.reshape(N, H, W // 2, 2 * C)
    return _s5(xv, cv)
